# baseline (speedup 1.0000x reference)
import functools

import jax
import jax.numpy as jnp
from jax import lax
from jax.experimental import pallas as pl
from jax.experimental.pallas import tpu as pltpu
from jax.experimental.pallas import tpu_sc as plsc

_FIELD_DIM = 38462
_F = 26
_B = 16384
_NC = 2
_NS = 16
_NW = _NC * _NS
_BW = _B // _NW
_L = 16
_CHUNK = 512
_QPF = _BW // _CHUNK

_TOT_PAD = 1000064
_FH = _F // 2
_PH = 500096
_P1B = 499968
_PS = 31232
_PS_LAST = _PH - (_NS - 1) * _PS

_mesh = plsc.VectorSubcoreMesh(core_axis_name="c", subcore_axis_name="s")


@functools.partial(
    pl.kernel,
    mesh=_mesh,
    compiler_params=pltpu.CompilerParams(needs_layout_passes=False),
    out_type=jax.ShapeDtypeStruct((_B,), jnp.float32),
    scratch_types=[
        pltpu.VMEM((_F, _BW), jnp.int32),
        pltpu.VMEM((_F * _BW,), jnp.int32),
        pltpu.VMEM((_F * _BW,), jnp.float32),
        pltpu.VMEM((_BW,), jnp.float32),
        pltpu.VMEM((_L,), jnp.float32),
        pltpu.VMEM_SHARED((_TOT_PAD,), jnp.float32),
        pltpu.SemaphoreType.DMA,
        pltpu.SemaphoreType.DMA,
    ],
)
def _embed_sum(xT, wt, bias, out, xb_v, idx_v, g_v, o_v, bias_v, tb_s, sem, sem2):
    c = lax.axis_index("c")
    s = lax.axis_index("s")
    wid = s * _NC + c
    base = wid * _BW

    def stage(phase_base, start):
        off = pl.multiple_of(phase_base + s * _PS, 128)
        off_l = pl.multiple_of(phase_base + (_NS - 1) * _PS, 128)

        @pl.when(s < _NS - 1)
        def _():
            cp = pltpu.make_async_copy(
                wt.at[0, pl.ds(off, _PS)], tb_s.at[pl.ds(off, _PS)], sem2
            )
            cp.start() if start else cp.wait()

        @pl.when(s == _NS - 1)
        def _():
            cp = pltpu.make_async_copy(
                wt.at[0, pl.ds(off_l, _PS_LAST)],
                tb_s.at[pl.ds(off_l, _PS_LAST)],
                sem2,
            )
            cp.start() if start else cp.wait()

    stage(0, True)
    stage(_P1B, True)

    pltpu.sync_copy(bias.at[pl.ds(0, 1)], bias_v.at[pl.ds(0, 1)])
    pltpu.sync_copy(xT.at[:, pl.ds(base, _BW)], xb_v)

    def per_field(f, _):
        off = f * _FIELD_DIM

        def build(j, _):
            for u in range(4):
                o = (j * 4 + u) * _L
                idx_v[pl.ds(f * _BW + o, _L)] = xb_v[f, pl.ds(o, _L)] + off
            return 0

        lax.fori_loop(0, _BW // (_L * 4), build, 0)
        return 0

    lax.fori_loop(0, _F, per_field, 0)

    def fire_field(f, _):
        def fire(q, _):
            qs = pl.ds(f * _BW + q * _CHUNK, _CHUNK)
            pltpu.make_async_copy(tb_s.at[idx_v.at[qs]], g_v.at[qs], sem).start()
            return 0

        lax.fori_loop(0, _QPF, fire, 0)
        return 0

    stage(0, False)
    plsc.subcore_barrier()
    lax.fori_loop(0, _FH, fire_field, 0)

    stage(_P1B, False)
    plsc.subcore_barrier()
    lax.fori_loop(_FH, _F, fire_field, 0)

    pltpu.make_async_copy(wt.at[0, pl.ds(0, _F * _BW)], g_v, sem).wait()

    bias_s = bias_v[pl.ds(0, _L)][0]
    _lanes = jax.lax.iota(jnp.int32, 16)
    _zeros = jnp.zeros((_L,), jnp.int32)

    def reduce16(j, _):
        acc = jnp.zeros((_L,), jnp.float32) + bias_s
        for f in range(_F):
            acc = acc + g_v[pl.ds(f * _BW + j * _L, _L)]
        o_v[pl.ds(j * _L, _L)] = acc
        return 0

    lax.fori_loop(0, _BW // _L, reduce16, 0)

    pltpu.sync_copy(o_v, out.at[pl.ds(base, _BW)])


def kernel(x, fc_weight, bias):
    y = _embed_sum(x.T, fc_weight.T, bias)
    return y.reshape(_B, 1)


# speedup vs baseline: 1.0136x; 1.0136x over previous
import functools

import jax
import jax.numpy as jnp
from jax import lax
from jax.experimental import pallas as pl
from jax.experimental.pallas import tpu as pltpu
from jax.experimental.pallas import tpu_sc as plsc

_FIELD_DIM = 38462
_F = 26
_B = 16384
_NC = 2
_NS = 16
_NW = _NC * _NS
_BW = _B // _NW
_L = 16
_CHUNK = 512
_QPF = _BW // _CHUNK

_TOT_PAD = 1000064
_FH = _F // 2
_PH = 500096
_P1B = 499968
_PS = 31232
_PS_LAST = _PH - (_NS - 1) * _PS

_mesh = plsc.VectorSubcoreMesh(core_axis_name="c", subcore_axis_name="s")


@functools.partial(
    pl.kernel,
    mesh=_mesh,
    compiler_params=pltpu.CompilerParams(needs_layout_passes=False),
    out_type=jax.ShapeDtypeStruct((_B,), jnp.float32),
    scratch_types=[
        pltpu.VMEM((_F, _BW), jnp.int32),
        pltpu.VMEM((_F * _BW,), jnp.int32),
        pltpu.VMEM((_F * _BW,), jnp.float32),
        pltpu.VMEM((_BW,), jnp.float32),
        pltpu.VMEM((_L,), jnp.float32),
        pltpu.VMEM_SHARED((_TOT_PAD,), jnp.float32),
        pltpu.SemaphoreType.DMA,
        pltpu.SemaphoreType.DMA,
    ],
)
def _embed_sum(xT, wt, bias, out, xb_v, idx_v, g_v, o_v, bias_v, tb_s, sem, sem2):
    c = lax.axis_index("c")
    s = lax.axis_index("s")
    wid = s * _NC + c
    base = wid * _BW

    def stage(phase_base, start):
        off = pl.multiple_of(phase_base + s * _PS, 128)
        off_l = pl.multiple_of(phase_base + (_NS - 1) * _PS, 128)

        @pl.when(s < _NS - 1)
        def _():
            cp = pltpu.make_async_copy(
                wt.at[0, pl.ds(off, _PS)], tb_s.at[pl.ds(off, _PS)], sem2
            )
            cp.start() if start else cp.wait()

        @pl.when(s == _NS - 1)
        def _():
            cp = pltpu.make_async_copy(
                wt.at[0, pl.ds(off_l, _PS_LAST)],
                tb_s.at[pl.ds(off_l, _PS_LAST)],
                sem2,
            )
            cp.start() if start else cp.wait()

    stage(0, True)
    stage(_P1B, True)

    pltpu.sync_copy(bias.at[pl.ds(0, 1)], bias_v.at[pl.ds(0, 1)])
    pltpu.sync_copy(xT.at[:, pl.ds(base, _BW)], xb_v)

    def build_fire_field(f, _):
        off = f * _FIELD_DIM

        def build(j, _):
            for u in range(8):
                o = (j * 8 + u) * _L
                idx_v[pl.ds(f * _BW + o, _L)] = xb_v[f, pl.ds(o, _L)] + off
            return 0

        lax.fori_loop(0, _BW // (_L * 8), build, 0)

        def fire(q, _):
            qs = pl.ds(f * _BW + q * _CHUNK, _CHUNK)
            pltpu.make_async_copy(tb_s.at[idx_v.at[qs]], g_v.at[qs], sem).start()
            return 0

        lax.fori_loop(0, _QPF, fire, 0)
        return 0

    stage(0, False)
    plsc.subcore_barrier()
    lax.fori_loop(0, _FH, build_fire_field, 0)

    stage(_P1B, False)
    plsc.subcore_barrier()
    lax.fori_loop(_FH, _F, build_fire_field, 0)

    pltpu.make_async_copy(wt.at[0, pl.ds(0, _F * _BW)], g_v, sem).wait()

    bias_s = bias_v[pl.ds(0, _L)][0]
    _lanes = jax.lax.iota(jnp.int32, 16)
    _zeros = jnp.zeros((_L,), jnp.int32)

    def reduce16(j, _):
        acc = jnp.zeros((_L,), jnp.float32) + bias_s
        for f in range(_F):
            acc = acc + g_v[pl.ds(f * _BW + j * _L, _L)]
        o_v[pl.ds(j * _L, _L)] = acc
        return 0

    lax.fori_loop(0, _BW // _L, reduce16, 0)

    pltpu.sync_copy(o_v, out.at[pl.ds(base, _BW)])


def kernel(x, fc_weight, bias):
    y = _embed_sum(x.T, fc_weight.T, bias)
    return y.reshape(_B, 1)


# speedup vs baseline: 1.0243x; 1.0106x over previous
import functools

import jax
import jax.numpy as jnp
from jax import lax
from jax.experimental import pallas as pl
from jax.experimental.pallas import tpu as pltpu
from jax.experimental.pallas import tpu_sc as plsc

_FIELD_DIM = 38462
_F = 26
_B = 16384
_NC = 2
_NS = 16
_NW = _NC * _NS
_BW = _B // _NW
_L = 16
_CHUNK = 512
_QPF = _BW // _CHUNK

_TOT_PAD = 1000064
_FH = _F // 2
_PH = 500096
_P1B = 499968
_PS = 31232
_PS_LAST = _PH - (_NS - 1) * _PS

_mesh = plsc.VectorSubcoreMesh(core_axis_name="c", subcore_axis_name="s")


@functools.partial(
    pl.kernel,
    mesh=_mesh,
    compiler_params=pltpu.CompilerParams(needs_layout_passes=False),
    out_type=jax.ShapeDtypeStruct((_B,), jnp.float32),
    scratch_types=[
        pltpu.VMEM((_F, _BW), jnp.int32),
        pltpu.VMEM((_F * _BW,), jnp.int32),
        pltpu.VMEM((_F * _BW,), jnp.float32),
        pltpu.VMEM((_BW,), jnp.float32),
        pltpu.VMEM((_L,), jnp.float32),
        pltpu.VMEM_SHARED((_TOT_PAD,), jnp.float32),
        pltpu.SemaphoreType.DMA,
        pltpu.SemaphoreType.DMA,
        pltpu.SemaphoreType.DMA,
    ],
)
def _embed_sum(xT, wt, bias, out, xb_v, idx_v, g_v, o_v, bias_v, tb_s, sem, sem2, sem3):
    c = lax.axis_index("c")
    s = lax.axis_index("s")
    wid = s * _NC + c
    base = wid * _BW

    def stage(phase_base, start):
        off = pl.multiple_of(phase_base + s * _PS, 128)
        off_l = pl.multiple_of(phase_base + (_NS - 1) * _PS, 128)

        @pl.when(s < _NS - 1)
        def _():
            cp = pltpu.make_async_copy(
                wt.at[0, pl.ds(off, _PS)], tb_s.at[pl.ds(off, _PS)], sem2
            )
            cp.start() if start else cp.wait()

        @pl.when(s == _NS - 1)
        def _():
            cp = pltpu.make_async_copy(
                wt.at[0, pl.ds(off_l, _PS_LAST)],
                tb_s.at[pl.ds(off_l, _PS_LAST)],
                sem2,
            )
            cp.start() if start else cp.wait()

    stage(0, True)
    stage(_P1B, True)

    pltpu.sync_copy(bias.at[pl.ds(0, 1)], bias_v.at[pl.ds(0, 1)])
    pltpu.sync_copy(xT.at[:, pl.ds(base, _BW)], xb_v)

    def build_fire_field(f, sem_f):
        off = f * _FIELD_DIM

        def build(j, _):
            for u in range(8):
                o = (j * 8 + u) * _L
                idx_v[pl.ds(f * _BW + o, _L)] = xb_v[f, pl.ds(o, _L)] + off
            return 0

        lax.fori_loop(0, _BW // (_L * 8), build, 0)

        def fire(q, _):
            qs = pl.ds(f * _BW + q * _CHUNK, _CHUNK)
            pltpu.make_async_copy(tb_s.at[idx_v.at[qs]], g_v.at[qs], sem_f).start()
            return 0

        lax.fori_loop(0, _QPF, fire, 0)
        return 0

    stage(0, False)
    plsc.subcore_barrier()
    lax.fori_loop(0, _FH, lambda f, _: build_fire_field(f, sem), 0)

    stage(_P1B, False)
    plsc.subcore_barrier()
    lax.fori_loop(_FH, _F, lambda f, _: build_fire_field(f, sem3), 0)

    bias_s = bias_v[pl.ds(0, _L)][0]

    halfA = pl.ds(0, _FH * _BW)
    pltpu.make_async_copy(wt.at[0, halfA], g_v.at[halfA], sem).wait()

    def reduceA(j, _):
        acc = jnp.zeros((_L,), jnp.float32) + bias_s
        for f in range(_FH):
            acc = acc + g_v[pl.ds(f * _BW + j * _L, _L)]
        o_v[pl.ds(j * _L, _L)] = acc
        return 0

    lax.fori_loop(0, _BW // _L, reduceA, 0)

    halfB = pl.ds(0, (_F - _FH) * _BW)
    pltpu.make_async_copy(wt.at[0, halfB], g_v.at[halfB], sem3).wait()

    def reduceB(j, _):
        acc = o_v[pl.ds(j * _L, _L)]
        for f in range(_FH, _F):
            acc = acc + g_v[pl.ds(f * _BW + j * _L, _L)]
        o_v[pl.ds(j * _L, _L)] = acc
        return 0

    lax.fori_loop(0, _BW // _L, reduceB, 0)

    pltpu.sync_copy(o_v, out.at[pl.ds(base, _BW)])


def kernel(x, fc_weight, bias):
    y = _embed_sum(x.T, fc_weight.T, bias)
    return y.reshape(_B, 1)


# speedup vs baseline: 1.0256x; 1.0013x over previous
import functools

import jax
import jax.numpy as jnp
from jax import lax
from jax.experimental import pallas as pl
from jax.experimental.pallas import tpu as pltpu
from jax.experimental.pallas import tpu_sc as plsc

_FIELD_DIM = 38462
_F = 26
_B = 16384
_NC = 2
_NS = 16
_NW = _NC * _NS
_BW = _B // _NW
_L = 16
_CHUNK = 512
_QPF = _BW // _CHUNK

_TOT_PAD = 1000064
_FH = _F // 2
_PH = 500096
_P1B = 499968
_PS = 31232
_PS_LAST = _PH - (_NS - 1) * _PS

_mesh = plsc.VectorSubcoreMesh(core_axis_name="c", subcore_axis_name="s")


@functools.partial(
    pl.kernel,
    mesh=_mesh,
    compiler_params=pltpu.CompilerParams(needs_layout_passes=False),
    out_type=jax.ShapeDtypeStruct((_B,), jnp.float32),
    scratch_types=[
        pltpu.VMEM((_F, _BW), jnp.int32),
        pltpu.VMEM((_F * _BW,), jnp.int32),
        pltpu.VMEM((_F * _BW,), jnp.float32),
        pltpu.VMEM((_BW,), jnp.float32),
        pltpu.VMEM((_L,), jnp.float32),
        pltpu.VMEM_SHARED((_TOT_PAD,), jnp.float32),
        pltpu.SemaphoreType.DMA,
        pltpu.SemaphoreType.DMA,
        pltpu.SemaphoreType.DMA,
        pltpu.SemaphoreType.DMA,
    ],
)
def _embed_sum(
    xT, wt, bias, out, xb_v, idx_v, g_v, o_v, bias_v, tb_s, sem, sem2, sem3, sem4
):
    c = lax.axis_index("c")
    s = lax.axis_index("s")
    wid = s * _NC + c
    base = wid * _BW

    def stage(phase_base, sem_p, start):
        off = pl.multiple_of(phase_base + s * _PS, 128)
        off_l = pl.multiple_of(phase_base + (_NS - 1) * _PS, 128)

        @pl.when(s < _NS - 1)
        def _():
            cp = pltpu.make_async_copy(
                wt.at[0, pl.ds(off, _PS)], tb_s.at[pl.ds(off, _PS)], sem_p
            )
            cp.start() if start else cp.wait()

        @pl.when(s == _NS - 1)
        def _():
            cp = pltpu.make_async_copy(
                wt.at[0, pl.ds(off_l, _PS_LAST)],
                tb_s.at[pl.ds(off_l, _PS_LAST)],
                sem_p,
            )
            cp.start() if start else cp.wait()

    stage(0, sem2, True)
    stage(_P1B, sem4, True)

    pltpu.sync_copy(bias.at[pl.ds(0, 1)], bias_v.at[pl.ds(0, 1)])
    pltpu.sync_copy(xT.at[:, pl.ds(base, _BW)], xb_v)

    def build_fire_field(f, sem_f):
        off = f * _FIELD_DIM

        def build(j, _):
            for u in range(8):
                o = (j * 8 + u) * _L
                idx_v[pl.ds(f * _BW + o, _L)] = xb_v[f, pl.ds(o, _L)] + off
            return 0

        lax.fori_loop(0, _BW // (_L * 8), build, 0)

        def fire(q, _):
            qs = pl.ds(f * _BW + q * _CHUNK, _CHUNK)
            pltpu.make_async_copy(tb_s.at[idx_v.at[qs]], g_v.at[qs], sem_f).start()
            return 0

        lax.fori_loop(0, _QPF, fire, 0)
        return 0

    stage(0, sem2, False)
    plsc.subcore_barrier()
    lax.fori_loop(0, _FH, lambda f, _: build_fire_field(f, sem), 0)

    stage(_P1B, sem4, False)
    plsc.subcore_barrier()
    lax.fori_loop(_FH, _F, lambda f, _: build_fire_field(f, sem3), 0)

    bias_s = bias_v[pl.ds(0, _L)][0]

    halfA = pl.ds(0, _FH * _BW)
    pltpu.make_async_copy(wt.at[0, halfA], g_v.at[halfA], sem).wait()

    def reduceA(j, _):
        acc = jnp.zeros((_L,), jnp.float32) + bias_s
        for f in range(_FH):
            acc = acc + g_v[pl.ds(f * _BW + j * _L, _L)]
        o_v[pl.ds(j * _L, _L)] = acc
        return 0

    lax.fori_loop(0, _BW // _L, reduceA, 0)

    halfB = pl.ds(0, (_F - _FH) * _BW)
    pltpu.make_async_copy(wt.at[0, halfB], g_v.at[halfB], sem3).wait()

    def reduceB(j, _):
        acc = o_v[pl.ds(j * _L, _L)]
        for f in range(_FH, _F):
            acc = acc + g_v[pl.ds(f * _BW + j * _L, _L)]
        o_v[pl.ds(j * _L, _L)] = acc
        return 0

    lax.fori_loop(0, _BW // _L, reduceB, 0)

    pltpu.sync_copy(o_v, out.at[pl.ds(base, _BW)])


def kernel(x, fc_weight, bias):
    y = _embed_sum(x.T, fc_weight.T, bias)
    return y.reshape(_B, 1)
